# Initial kernel scaffold; baseline (speedup 1.0000x reference)
import jax
import jax.numpy as jnp
from jax.experimental import pallas as pl


def kernel(loc, depot, W_init, b_init, W_ne, b_ne, W_dep, b_dep, W_t1, b_t1, W_t2, b_t2):
    raise NotImplementedError("write your pallas kernel here")


# speedup vs baseline: 8.9712x; 8.9712x over previous
import functools

import jax
import jax.numpy as jnp
from jax import lax
from jax.experimental import pallas as pl
from jax.experimental.pallas import tpu as pltpu
from jax.experimental.pallas import tpu_sc as plsc

B, N, D, E = 2, 2048, 2, 256
K = 10
R = 256
NC, NS = 2, 16
NW = NC * NS
NODES = B * N
NODES_PER_W = NODES // NW
G = 8
GROUPS = NODES_PER_W // G


def _leaky(z):
  return jnp.where(z >= 0, z, 0.01 * z)


def _topk_f1_body(x_ref, xT_ref, x0_ref, wt1_ref, winitT_ref, binit_ref,
                  wneT_ref, bne_ref, bt1_ref, nb_ref, f1_ref):
  xi_x = x_ref[0, :, 0:1]
  xi_y = x_ref[0, :, 1:2]
  xj_x = xT_ref[0, 0:1, :]
  xj_y = xT_ref[0, 1:2, :]
  dx = xj_x - xi_x
  dy = xj_y - xi_y
  key = jnp.sqrt(dx * dx + dy * dy)
  iota = lax.broadcasted_iota(jnp.int32, (R, N), 1)

  dn = (((1,), (1,)), ((), ()))
  wc0T = lax.dot_general(winitT_ref[...], wt1_ref[...], dn,
                         preferred_element_type=jnp.float32)
  bc0 = lax.dot_general(binit_ref[...], wt1_ref[...], dn,
                        preferred_element_type=jnp.float32) + bt1_ref[...]
  wcnT = lax.dot_general(wneT_ref[...], wt1_ref[...], dn,
                         preferred_element_type=jnp.float32)
  bcn = lax.dot_general(bne_ref[...], wt1_ref[...], dn,
                        preferred_element_type=jnp.float32) + bt1_ref[...]

  z0 = xi_x * wc0T[0:1, :] + xi_y * wc0T[1:2, :] + bc0
  acc = _leaky(z0)

  cols = []
  for _ in range(K):
    m = jnp.min(key, axis=1, keepdims=True)
    idx = jnp.min(jnp.where(key == m, iota, N), axis=1, keepdims=True)
    cols.append(idx)
    onehot = iota == idx
    key = jnp.where(onehot, jnp.inf, key)
    nbxy = lax.dot_general(onehot.astype(jnp.float32), x0_ref[...],
                           (((1,), (0,)), ((), ())),
                           preferred_element_type=jnp.float32)
    dxn = nbxy[:, 0:1] - xi_x
    dyn = nbxy[:, 1:2] - xi_y
    acc = acc + _leaky(dxn * wcnT[0:1, :] + dyn * wcnT[1:2, :] + bcn)

  nb_ref[0] = jnp.concatenate(cols, axis=1)
  f1_ref[0] = acc


def _gl_dep_body(f1_ref, wt2_ref, bt2_ref, depot_ref, wdep_ref, bdep_ref,
                 gl_ref, dep_ref):
  dn = (((1,), (1,)), ((), ()))
  g = lax.dot_general(f1_ref[...], wt2_ref[...], dn,
                      preferred_element_type=jnp.float32) + bt2_ref[...]
  gl_ref[...] = _leaky(g)
  dep_ref[...] = lax.dot_general(depot_ref[...], wdep_ref[...], dn,
                                 preferred_element_type=jnp.float32) + bdep_ref[...]


def _sc_gather_sum_body(idx_hbm, table_hbm, out_hbm, idx_v, rows_v, out_v, sem):
  wid = lax.axis_index("s") * NC + lax.axis_index("c")
  base_node = wid * NODES_PER_W

  def group(g, carry):
    node0 = base_node + g * G
    pltpu.sync_copy(idx_hbm.at[pl.ds(node0 * K, G * K)], idx_v)
    pltpu.async_copy(table_hbm.at[idx_v], rows_v, sem).wait()
    for i in range(G):
      for c in range(E // 16):
        acc = rows_v[i * K + 0, pl.ds(c * 16, 16)]
        for k in range(1, K):
          acc = acc + rows_v[i * K + k, pl.ds(c * 16, 16)]
        out_v[i, pl.ds(c * 16, 16)] = acc
    pltpu.sync_copy(out_v, out_hbm.at[pl.ds(node0, G)])
    return carry

  lax.fori_loop(0, GROUPS, group, 0)


def _mean_body(f2_ref, dep_ref, mean_ref):
  b = pl.program_id(0)
  s = jnp.sum(f2_ref[0], axis=0, keepdims=True) + dep_ref[pl.ds(b, 1), :]
  mean_ref[pl.ds(b, 1), :] = s / float(N + 1)


def _make_calls(interpret=False):
  topk_f1 = pl.pallas_call(
      _topk_f1_body,
      grid=(B, N // R),
      in_specs=[
          pl.BlockSpec((1, R, D), lambda b, t: (b, t, 0)),
          pl.BlockSpec((1, D, N), lambda b, t: (b, 0, 0)),
          pl.BlockSpec((N, D), lambda b, t: (0, 0)),
          pl.BlockSpec((E, E), lambda b, t: (0, 0)),
          pl.BlockSpec((D, E), lambda b, t: (0, 0)),
          pl.BlockSpec((1, E), lambda b, t: (0, 0)),
          pl.BlockSpec((D, E), lambda b, t: (0, 0)),
          pl.BlockSpec((1, E), lambda b, t: (0, 0)),
          pl.BlockSpec((1, E), lambda b, t: (0, 0)),
      ],
      out_specs=[
          pl.BlockSpec((1, R, K), lambda b, t: (b, t, 0)),
          pl.BlockSpec((1, R, E), lambda b, t: (b, t, 0)),
      ],
      out_shape=[
          jax.ShapeDtypeStruct((B, N, K), jnp.int32),
          jax.ShapeDtypeStruct((B, N, E), jnp.float32),
      ],
      interpret=interpret,
  )

  gl_dep = pl.pallas_call(
      _gl_dep_body,
      out_shape=[
          jax.ShapeDtypeStruct((N, E), jnp.float32),
          jax.ShapeDtypeStruct((B, E), jnp.float32),
      ],
      interpret=interpret,
  )

  mean = pl.pallas_call(
      _mean_body,
      grid=(B,),
      in_specs=[
          pl.BlockSpec((1, N, E), lambda b: (b, 0, 0)),
          pl.BlockSpec((B, E), lambda b: (0, 0)),
      ],
      out_specs=pl.BlockSpec((B, E), lambda b: (0, 0)),
      out_shape=jax.ShapeDtypeStruct((B, E), jnp.float32),
      interpret=interpret,
  )
  return topk_f1, gl_dep, mean


_TOPK_F1, _GL_DEP, _MEAN = _make_calls()


@functools.cache
def _sc_gather_sum_call():
  return functools.partial(
      pl.kernel,
      out_type=jax.ShapeDtypeStruct((NODES, E), jnp.float32),
      mesh=plsc.VectorSubcoreMesh(core_axis_name="c", subcore_axis_name="s"),
      scratch_types=[
          pltpu.VMEM((G * K,), jnp.int32),
          pltpu.VMEM((G * K, E), jnp.float32),
          pltpu.VMEM((G, E), jnp.float32),
          pltpu.SemaphoreType.DMA,
      ],
  )(_sc_gather_sum_body)


@jax.jit
def kernel(loc, depot, W_init, b_init, W_ne, b_ne, W_dep, b_dep,
           W_t1, b_t1, W_t2, b_t2):
  x = loc
  xT = jnp.transpose(x, (0, 2, 1))
  nb, f1 = _TOPK_F1(x, xT, x[0], W_t1, W_init.T, b_init[None, :],
                    W_ne.T, b_ne[None, :], b_t1[None, :])
  gl, dep = _GL_DEP(f1[0], W_t2, b_t2[None, :], depot[:, 0, :], W_dep,
                    b_dep[None, :])
  f2 = _sc_gather_sum_call()(nb.reshape(NODES * K), gl).reshape(B, N, E)
  mean = _MEAN(f2, dep)
  h = jnp.concatenate([dep[:, None, :], f2], axis=1)
  return (h, mean)


# speedup vs baseline: 10.0983x; 1.1256x over previous
import functools

import jax
import jax.numpy as jnp
from jax import lax
from jax.experimental import pallas as pl
from jax.experimental.pallas import tpu as pltpu
from jax.experimental.pallas import tpu_sc as plsc

B, N, D, E = 2, 2048, 2, 256
K = 10
R = 256
NC, NS = 2, 16
NW = NC * NS
NODES = B * N
NODES_PER_W = NODES // NW
G = 8
GROUPS = NODES_PER_W // G


def _leaky(z):
  return jnp.where(z >= 0, z, 0.01 * z)


def _topk_f1_body(x_ref, xT_ref, x0_ref, wt1_ref, winitT_ref, binit_ref,
                  wneT_ref, bne_ref, bt1_ref, nb_ref, f1_ref):
  xi_x = x_ref[0, :, 0:1]
  xi_y = x_ref[0, :, 1:2]
  xj_x = xT_ref[0, 0:1, :]
  xj_y = xT_ref[0, 1:2, :]
  dx = xj_x - xi_x
  dy = xj_y - xi_y
  key = jnp.sqrt(dx * dx + dy * dy)
  iota = lax.broadcasted_iota(jnp.int32, (R, N), 1)

  dn = (((1,), (1,)), ((), ()))
  wc0T = lax.dot_general(winitT_ref[...], wt1_ref[...], dn,
                         preferred_element_type=jnp.float32)
  bc0 = lax.dot_general(binit_ref[...], wt1_ref[...], dn,
                        preferred_element_type=jnp.float32) + bt1_ref[...]
  wcnT = lax.dot_general(wneT_ref[...], wt1_ref[...], dn,
                         preferred_element_type=jnp.float32)
  bcn = lax.dot_general(bne_ref[...], wt1_ref[...], dn,
                        preferred_element_type=jnp.float32) + bt1_ref[...]

  z0 = xi_x * wc0T[0:1, :] + xi_y * wc0T[1:2, :] + bc0
  acc = _leaky(z0)

  cols = []
  for _ in range(K):
    m = jnp.min(key, axis=1, keepdims=True)
    idx = jnp.min(jnp.where(key == m, iota, N), axis=1, keepdims=True)
    cols.append(idx)
    onehot = iota == idx
    key = jnp.where(onehot, jnp.inf, key)
    nbxy = lax.dot_general(onehot.astype(jnp.float32), x0_ref[...],
                           (((1,), (0,)), ((), ())),
                           preferred_element_type=jnp.float32)
    dxn = nbxy[:, 0:1] - xi_x
    dyn = nbxy[:, 1:2] - xi_y
    acc = acc + _leaky(dxn * wcnT[0:1, :] + dyn * wcnT[1:2, :] + bcn)

  nb_ref[0] = jnp.concatenate(cols, axis=1)
  f1_ref[0] = acc


def _gl_dep_body(f1_ref, wt2_ref, bt2_ref, depot_ref, wdep_ref, bdep_ref,
                 gl_ref, dep_ref):
  dn = (((1,), (1,)), ((), ()))
  g = lax.dot_general(f1_ref[...], wt2_ref[...], dn,
                      preferred_element_type=jnp.float32) + bt2_ref[...]
  gl_ref[...] = _leaky(g)
  dep_ref[...] = lax.dot_general(depot_ref[...], wdep_ref[...], dn,
                                 preferred_element_type=jnp.float32) + bdep_ref[...]


def _sc_gather_sum_body(idx_hbm, table_hbm, out_hbm, idx_v, rows_v, out_v,
                        gsem, osem):
  wid = lax.axis_index("s") * NC + lax.axis_index("c")
  base_node = wid * NODES_PER_W

  pltpu.sync_copy(idx_hbm.at[wid], idx_v)
  pltpu.async_copy(table_hbm.at[idx_v.at[0]], rows_v.at[0], gsem)

  def pair(i, carry):
    for b in range(2):
      g = 2 * i + b
      nxt = g + 1
      pltpu.make_async_copy(table_hbm.at[idx_v.at[g]], rows_v.at[b],
                            gsem).wait()

      @pl.when(nxt < GROUPS)
      def _():
        pltpu.async_copy(table_hbm.at[idx_v.at[nxt]], rows_v.at[1 - b], gsem)

      @pl.when(g >= 2)
      def _():
        pltpu.make_async_copy(out_v.at[b], out_hbm.at[pl.ds(base_node, G)],
                              osem).wait()

      for i2 in range(G):
        for c in range(E // 16):
          acc = rows_v[b, i2 * K + 0, pl.ds(c * 16, 16)]
          for k in range(1, K):
            acc = acc + rows_v[b, i2 * K + k, pl.ds(c * 16, 16)]
          out_v[b, i2, pl.ds(c * 16, 16)] = acc
      pltpu.async_copy(out_v.at[b], out_hbm.at[pl.ds(base_node + g * G, G)],
                       osem)
    return carry

  lax.fori_loop(0, GROUPS // 2, pair, 0)
  for b in range(2):
    pltpu.make_async_copy(out_v.at[b], out_hbm.at[pl.ds(base_node, G)],
                          osem).wait()


def _mean_body(f2_ref, dep_ref, mean_ref):
  b = pl.program_id(0)
  s = jnp.sum(f2_ref[0], axis=0, keepdims=True) + dep_ref[pl.ds(b, 1), :]
  mean_ref[pl.ds(b, 1), :] = s / float(N + 1)


def _make_calls(interpret=False):
  topk_f1 = pl.pallas_call(
      _topk_f1_body,
      grid=(B, N // R),
      in_specs=[
          pl.BlockSpec((1, R, D), lambda b, t: (b, t, 0)),
          pl.BlockSpec((1, D, N), lambda b, t: (b, 0, 0)),
          pl.BlockSpec((N, D), lambda b, t: (0, 0)),
          pl.BlockSpec((E, E), lambda b, t: (0, 0)),
          pl.BlockSpec((D, E), lambda b, t: (0, 0)),
          pl.BlockSpec((1, E), lambda b, t: (0, 0)),
          pl.BlockSpec((D, E), lambda b, t: (0, 0)),
          pl.BlockSpec((1, E), lambda b, t: (0, 0)),
          pl.BlockSpec((1, E), lambda b, t: (0, 0)),
      ],
      out_specs=[
          pl.BlockSpec((1, R, K), lambda b, t: (b, t, 0)),
          pl.BlockSpec((1, R, E), lambda b, t: (b, t, 0)),
      ],
      out_shape=[
          jax.ShapeDtypeStruct((B, N, K), jnp.int32),
          jax.ShapeDtypeStruct((B, N, E), jnp.float32),
      ],
      interpret=interpret,
  )

  gl_dep = pl.pallas_call(
      _gl_dep_body,
      out_shape=[
          jax.ShapeDtypeStruct((N, E), jnp.float32),
          jax.ShapeDtypeStruct((B, E), jnp.float32),
      ],
      interpret=interpret,
  )

  mean = pl.pallas_call(
      _mean_body,
      grid=(B,),
      in_specs=[
          pl.BlockSpec((1, N, E), lambda b: (b, 0, 0)),
          pl.BlockSpec((B, E), lambda b: (0, 0)),
      ],
      out_specs=pl.BlockSpec((B, E), lambda b: (0, 0)),
      out_shape=jax.ShapeDtypeStruct((B, E), jnp.float32),
      interpret=interpret,
  )
  return topk_f1, gl_dep, mean


_TOPK_F1, _GL_DEP, _MEAN = _make_calls()


@functools.cache
def _sc_gather_sum_call():
  return functools.partial(
      pl.kernel,
      out_type=jax.ShapeDtypeStruct((NODES, E), jnp.float32),
      mesh=plsc.VectorSubcoreMesh(core_axis_name="c", subcore_axis_name="s"),
      scratch_types=[
          pltpu.VMEM((GROUPS, G * K), jnp.int32),
          pltpu.VMEM((2, G * K, E), jnp.float32),
          pltpu.VMEM((2, G, E), jnp.float32),
          pltpu.SemaphoreType.DMA,
          pltpu.SemaphoreType.DMA,
      ],
  )(_sc_gather_sum_body)


@jax.jit
def kernel(loc, depot, W_init, b_init, W_ne, b_ne, W_dep, b_dep,
           W_t1, b_t1, W_t2, b_t2):
  x = loc
  xT = jnp.transpose(x, (0, 2, 1))
  nb, f1 = _TOPK_F1(x, xT, x[0], W_t1, W_init.T, b_init[None, :],
                    W_ne.T, b_ne[None, :], b_t1[None, :])
  gl, dep = _GL_DEP(f1[0], W_t2, b_t2[None, :], depot[:, 0, :], W_dep,
                    b_dep[None, :])
  f2 = _sc_gather_sum_call()(nb.reshape(NW, GROUPS, G * K), gl).reshape(B, N, E)
  mean = _MEAN(f2, dep)
  h = jnp.concatenate([dep[:, None, :], f2], axis=1)
  return (h, mean)


# speedup vs baseline: 10.6933x; 1.0589x over previous
import functools

import jax
import jax.numpy as jnp
from jax import lax
from jax.experimental import pallas as pl
from jax.experimental.pallas import tpu as pltpu
from jax.experimental.pallas import tpu_sc as plsc

B, N, D, E = 2, 2048, 2, 256
K = 10
R = 256
NC, NS = 2, 16
NW = NC * NS
NODES = B * N
NODES_PER_W = NODES // NW
G = 8
GROUPS = NODES_PER_W // G


def _leaky(z):
  return jnp.where(z >= 0, z, 0.01 * z)


def _topk_f1_body(x_ref, xT_ref, x0_ref, wt1_ref, winitT_ref, binit_ref,
                  wneT_ref, bne_ref, bt1_ref, nb_ref, f1_ref):
  xi_x = x_ref[0, :, 0:1]
  xi_y = x_ref[0, :, 1:2]
  xj_x = xT_ref[0, 0:1, :]
  xj_y = xT_ref[0, 1:2, :]
  dx = xj_x - xi_x
  dy = xj_y - xi_y
  key = jnp.sqrt(dx * dx + dy * dy)
  fiota = lax.broadcasted_iota(jnp.int32, (R, N), 1).astype(jnp.float32)

  dn = (((1,), (1,)), ((), ()))
  wc0T = lax.dot_general(winitT_ref[...], wt1_ref[...], dn,
                         preferred_element_type=jnp.float32)
  bc0 = lax.dot_general(binit_ref[...], wt1_ref[...], dn,
                        preferred_element_type=jnp.float32) + bt1_ref[...]
  wcnT = lax.dot_general(wneT_ref[...], wt1_ref[...], dn,
                         preferred_element_type=jnp.float32)
  bcn = lax.dot_general(bne_ref[...], wt1_ref[...], dn,
                        preferred_element_type=jnp.float32) + bt1_ref[...]

  z0 = xi_x * wc0T[0:1, :] + xi_y * wc0T[1:2, :] + bc0
  acc = _leaky(z0)

  cols = []
  for _ in range(K):
    m = jnp.min(key, axis=1, keepdims=True)
    idx = jnp.min(jnp.where(key == m, fiota, float(N)), axis=1, keepdims=True)
    cols.append(idx)
    onehot_f = jnp.where(fiota == idx, 1.0, 0.0)
    key = jnp.where(fiota == idx, jnp.inf, key)
    nbxy = lax.dot_general(onehot_f, x0_ref[...],
                           (((1,), (0,)), ((), ())),
                           preferred_element_type=jnp.float32)
    dxn = nbxy[:, 0:1] - xi_x
    dyn = nbxy[:, 1:2] - xi_y
    acc = acc + _leaky(dxn * wcnT[0:1, :] + dyn * wcnT[1:2, :] + bcn)

  nb_ref[0] = jnp.concatenate(cols, axis=1).astype(jnp.int32)
  f1_ref[0] = acc


def _gl_dep_body(f1_ref, wt2_ref, bt2_ref, depot_ref, wdep_ref, bdep_ref,
                 gl_ref, dep_ref):
  dn = (((1,), (1,)), ((), ()))
  g = lax.dot_general(f1_ref[...], wt2_ref[...], dn,
                      preferred_element_type=jnp.float32) + bt2_ref[...]
  gl_ref[...] = _leaky(g)
  dep_ref[...] = lax.dot_general(depot_ref[...], wdep_ref[...], dn,
                                 preferred_element_type=jnp.float32) + bdep_ref[...]


def _sc_gather_sum_body(idx_hbm, table_hbm, out_hbm, idx_v, rows_v, out_v,
                        gsem, osem):
  wid = lax.axis_index("s") * NC + lax.axis_index("c")
  base_node = wid * NODES_PER_W

  pltpu.sync_copy(idx_hbm.at[wid], idx_v)
  pltpu.async_copy(table_hbm.at[idx_v.at[0]], rows_v.at[0], gsem)

  def pair(i, carry):
    for b in range(2):
      g = 2 * i + b
      nxt = g + 1
      pltpu.make_async_copy(table_hbm.at[idx_v.at[g]], rows_v.at[b],
                            gsem).wait()

      @pl.when(nxt < GROUPS)
      def _():
        pltpu.async_copy(table_hbm.at[idx_v.at[nxt]], rows_v.at[1 - b], gsem)

      @pl.when(g >= 2)
      def _():
        pltpu.make_async_copy(out_v.at[b], out_hbm.at[pl.ds(base_node, G)],
                              osem).wait()

      for i2 in range(G):
        for c in range(E // 16):
          r = [rows_v[b, i2 * K + k, pl.ds(c * 16, 16)] for k in range(K)]
          s01, s23 = r[0] + r[1], r[2] + r[3]
          s45, s67 = r[4] + r[5], r[6] + r[7]
          s89 = r[8] + r[9]
          out_v[b, i2, pl.ds(c * 16, 16)] = ((s01 + s23) + (s45 + s67)) + s89
      pltpu.async_copy(out_v.at[b], out_hbm.at[pl.ds(base_node + g * G, G)],
                       osem)
    return carry

  lax.fori_loop(0, GROUPS // 2, pair, 0)
  for b in range(2):
    pltpu.make_async_copy(out_v.at[b], out_hbm.at[pl.ds(base_node, G)],
                          osem).wait()


def _mean_body(f2_ref, dep_ref, mean_ref):
  b = pl.program_id(0)
  s = jnp.sum(f2_ref[0], axis=0, keepdims=True) + dep_ref[pl.ds(b, 1), :]
  mean_ref[pl.ds(b, 1), :] = s / float(N + 1)


def _make_calls(interpret=False):
  topk_f1 = pl.pallas_call(
      _topk_f1_body,
      grid=(B, N // R),
      in_specs=[
          pl.BlockSpec((1, R, D), lambda b, t: (b, t, 0)),
          pl.BlockSpec((1, D, N), lambda b, t: (b, 0, 0)),
          pl.BlockSpec((N, D), lambda b, t: (0, 0)),
          pl.BlockSpec((E, E), lambda b, t: (0, 0)),
          pl.BlockSpec((D, E), lambda b, t: (0, 0)),
          pl.BlockSpec((1, E), lambda b, t: (0, 0)),
          pl.BlockSpec((D, E), lambda b, t: (0, 0)),
          pl.BlockSpec((1, E), lambda b, t: (0, 0)),
          pl.BlockSpec((1, E), lambda b, t: (0, 0)),
      ],
      out_specs=[
          pl.BlockSpec((1, R, K), lambda b, t: (b, t, 0)),
          pl.BlockSpec((1, R, E), lambda b, t: (b, t, 0)),
      ],
      out_shape=[
          jax.ShapeDtypeStruct((B, N, K), jnp.int32),
          jax.ShapeDtypeStruct((B, N, E), jnp.float32),
      ],
      interpret=interpret,
  )

  gl_dep = pl.pallas_call(
      _gl_dep_body,
      out_shape=[
          jax.ShapeDtypeStruct((N, E), jnp.float32),
          jax.ShapeDtypeStruct((B, E), jnp.float32),
      ],
      interpret=interpret,
  )

  mean = pl.pallas_call(
      _mean_body,
      grid=(B,),
      in_specs=[
          pl.BlockSpec((1, N, E), lambda b: (b, 0, 0)),
          pl.BlockSpec((B, E), lambda b: (0, 0)),
      ],
      out_specs=pl.BlockSpec((B, E), lambda b: (0, 0)),
      out_shape=jax.ShapeDtypeStruct((B, E), jnp.float32),
      interpret=interpret,
  )
  return topk_f1, gl_dep, mean


_TOPK_F1, _GL_DEP, _MEAN = _make_calls()


@functools.cache
def _sc_gather_sum_call():
  return functools.partial(
      pl.kernel,
      out_type=jax.ShapeDtypeStruct((NODES, E), jnp.float32),
      mesh=plsc.VectorSubcoreMesh(core_axis_name="c", subcore_axis_name="s"),
      scratch_types=[
          pltpu.VMEM((GROUPS, G * K), jnp.int32),
          pltpu.VMEM((2, G * K, E), jnp.float32),
          pltpu.VMEM((2, G, E), jnp.float32),
          pltpu.SemaphoreType.DMA,
          pltpu.SemaphoreType.DMA,
      ],
  )(_sc_gather_sum_body)


@jax.jit
def kernel(loc, depot, W_init, b_init, W_ne, b_ne, W_dep, b_dep,
           W_t1, b_t1, W_t2, b_t2):
  x = loc
  xT = jnp.transpose(x, (0, 2, 1))
  nb, f1 = _TOPK_F1(x, xT, x[0], W_t1, W_init.T, b_init[None, :],
                    W_ne.T, b_ne[None, :], b_t1[None, :])
  gl, dep = _GL_DEP(f1[0], W_t2, b_t2[None, :], depot[:, 0, :], W_dep,
                    b_dep[None, :])
  f2 = _sc_gather_sum_call()(nb.reshape(NW, GROUPS, G * K), gl).reshape(B, N, E)
  mean = _MEAN(f2, dep)
  h = jnp.concatenate([dep[:, None, :], f2], axis=1)
  return (h, mean)


# speedup vs baseline: 12.2352x; 1.1442x over previous
import functools

import jax
import jax.numpy as jnp
from jax import lax
from jax.experimental import pallas as pl
from jax.experimental.pallas import tpu as pltpu
from jax.experimental.pallas import tpu_sc as plsc

B, N, D, E = 2, 2048, 2, 256
K = 10
R = 256
NC, NS = 2, 16
NW = NC * NS
NODES = B * N
NSPLIT = 2048
T_E = 256
SC_NODES = NODES - NSPLIT
NPW_SC = SC_NODES // NW
G = 8
GROUPS = NPW_SC // G


def _leaky(z):
  return jnp.where(z >= 0, z, 0.01 * z)


def _topk_f1_body(x_ref, xT_ref, x0_ref, wt1_ref, winitT_ref, binit_ref,
                  wneT_ref, bne_ref, bt1_ref, nb_ref, f1_ref):
  xi_x = x_ref[0, :, 0:1]
  xi_y = x_ref[0, :, 1:2]
  xj_x = xT_ref[0, 0:1, :]
  xj_y = xT_ref[0, 1:2, :]
  dx = xj_x - xi_x
  dy = xj_y - xi_y
  key = jnp.sqrt(dx * dx + dy * dy)
  fiota = lax.broadcasted_iota(jnp.int32, (R, N), 1).astype(jnp.float32)

  dn = (((1,), (1,)), ((), ()))
  wc0T = lax.dot_general(winitT_ref[...], wt1_ref[...], dn,
                         preferred_element_type=jnp.float32)
  bc0 = lax.dot_general(binit_ref[...], wt1_ref[...], dn,
                        preferred_element_type=jnp.float32) + bt1_ref[...]
  wcnT = lax.dot_general(wneT_ref[...], wt1_ref[...], dn,
                         preferred_element_type=jnp.float32)
  bcn = lax.dot_general(bne_ref[...], wt1_ref[...], dn,
                        preferred_element_type=jnp.float32) + bt1_ref[...]

  z0 = xi_x * wc0T[0:1, :] + xi_y * wc0T[1:2, :] + bc0
  acc = _leaky(z0)

  cols = []
  for _ in range(K):
    m = jnp.min(key, axis=1, keepdims=True)
    idx = jnp.min(jnp.where(key == m, fiota, float(N)), axis=1, keepdims=True)
    cols.append(idx)
    onehot_f = jnp.where(fiota == idx, 1.0, 0.0)
    key = jnp.where(fiota == idx, jnp.inf, key)
    nbxy = lax.dot_general(onehot_f, x0_ref[...],
                           (((1,), (0,)), ((), ())),
                           preferred_element_type=jnp.float32)
    dxn = nbxy[:, 0:1] - xi_x
    dyn = nbxy[:, 1:2] - xi_y
    acc = acc + _leaky(dxn * wcnT[0:1, :] + dyn * wcnT[1:2, :] + bcn)

  nb_ref[0] = jnp.concatenate(cols, axis=1).astype(jnp.int32)
  f1_ref[0] = acc


def _gl_dep_body(f1_ref, wt2_ref, bt2_ref, depot_ref, wdep_ref, bdep_ref,
                 gl_ref, dep_ref):
  dn = (((1,), (1,)), ((), ()))
  g = lax.dot_general(f1_ref[...], wt2_ref[...], dn,
                      preferred_element_type=jnp.float32) + bt2_ref[...]
  gl_ref[...] = _leaky(g)
  dep_ref[...] = lax.dot_general(depot_ref[...], wdep_ref[...], dn,
                                 preferred_element_type=jnp.float32) + bdep_ref[...]


def _f2_onehot_body(nbf_ref, gl_ref, out_ref):
  nbk = nbf_ref[...]
  iota = lax.broadcasted_iota(jnp.int32, (T_E, N), 1)
  mask = jnp.where(iota == nbk[:, 0:1], 1.0, 0.0)
  for k in range(1, K):
    mask = mask + jnp.where(iota == nbk[:, k:k + 1], 1.0, 0.0)
  out_ref[...] = lax.dot_general(mask, gl_ref[...], (((1,), (0,)), ((), ())),
                                 preferred_element_type=jnp.float32)


def _sc_gather_sum_body(idx_hbm, table_hbm, out_hbm, idx_v, rows_v, out_v,
                        gsem, osem):
  wid = lax.axis_index("s") * NC + lax.axis_index("c")
  base_node = wid * NPW_SC

  pltpu.sync_copy(idx_hbm.at[wid], idx_v)
  pltpu.async_copy(table_hbm.at[idx_v.at[0]], rows_v.at[0], gsem)

  def pair(i, carry):
    for b in range(2):
      g = 2 * i + b
      nxt = g + 1
      pltpu.make_async_copy(table_hbm.at[idx_v.at[g]], rows_v.at[b],
                            gsem).wait()

      @pl.when(nxt < GROUPS)
      def _():
        pltpu.async_copy(table_hbm.at[idx_v.at[nxt]], rows_v.at[1 - b], gsem)

      @pl.when(g >= 2)
      def _():
        pltpu.make_async_copy(out_v.at[b], out_hbm.at[pl.ds(base_node, G)],
                              osem).wait()

      for i2 in range(G):
        for c in range(E // 16):
          r = [rows_v[b, i2 * K + k, pl.ds(c * 16, 16)] for k in range(K)]
          s01, s23 = r[0] + r[1], r[2] + r[3]
          s45, s67 = r[4] + r[5], r[6] + r[7]
          s89 = r[8] + r[9]
          out_v[b, i2, pl.ds(c * 16, 16)] = ((s01 + s23) + (s45 + s67)) + s89
      pltpu.async_copy(out_v.at[b], out_hbm.at[pl.ds(base_node + g * G, G)],
                       osem)
    return carry

  lax.fori_loop(0, GROUPS // 2, pair, 0)
  for b in range(2):
    pltpu.make_async_copy(out_v.at[b], out_hbm.at[pl.ds(base_node, G)],
                          osem).wait()


def _mean_body(f2_ref, dep_ref, mean_ref):
  b = pl.program_id(0)
  s = jnp.sum(f2_ref[0], axis=0, keepdims=True) + dep_ref[pl.ds(b, 1), :]
  mean_ref[pl.ds(b, 1), :] = s / float(N + 1)


def _make_calls(interpret=False):
  topk_f1 = pl.pallas_call(
      _topk_f1_body,
      grid=(B, N // R),
      in_specs=[
          pl.BlockSpec((1, R, D), lambda b, t: (b, t, 0)),
          pl.BlockSpec((1, D, N), lambda b, t: (b, 0, 0)),
          pl.BlockSpec((N, D), lambda b, t: (0, 0)),
          pl.BlockSpec((E, E), lambda b, t: (0, 0)),
          pl.BlockSpec((D, E), lambda b, t: (0, 0)),
          pl.BlockSpec((1, E), lambda b, t: (0, 0)),
          pl.BlockSpec((D, E), lambda b, t: (0, 0)),
          pl.BlockSpec((1, E), lambda b, t: (0, 0)),
          pl.BlockSpec((1, E), lambda b, t: (0, 0)),
      ],
      out_specs=[
          pl.BlockSpec((1, R, K), lambda b, t: (b, t, 0)),
          pl.BlockSpec((1, R, E), lambda b, t: (b, t, 0)),
      ],
      out_shape=[
          jax.ShapeDtypeStruct((B, N, K), jnp.int32),
          jax.ShapeDtypeStruct((B, N, E), jnp.float32),
      ],
      interpret=interpret,
  )

  f2_onehot = pl.pallas_call(
      _f2_onehot_body,
      grid=(NSPLIT // T_E,),
      in_specs=[
          pl.BlockSpec((T_E, K), lambda t: (t, 0)),
          pl.BlockSpec((N, E), lambda t: (0, 0)),
      ],
      out_specs=pl.BlockSpec((T_E, E), lambda t: (t, 0)),
      out_shape=jax.ShapeDtypeStruct((NSPLIT, E), jnp.float32),
      interpret=interpret,
  )

  gl_dep = pl.pallas_call(
      _gl_dep_body,
      out_shape=[
          jax.ShapeDtypeStruct((N, E), jnp.float32),
          jax.ShapeDtypeStruct((B, E), jnp.float32),
      ],
      interpret=interpret,
  )

  mean = pl.pallas_call(
      _mean_body,
      grid=(B,),
      in_specs=[
          pl.BlockSpec((1, N, E), lambda b: (b, 0, 0)),
          pl.BlockSpec((B, E), lambda b: (0, 0)),
      ],
      out_specs=pl.BlockSpec((B, E), lambda b: (0, 0)),
      out_shape=jax.ShapeDtypeStruct((B, E), jnp.float32),
      interpret=interpret,
  )
  return topk_f1, f2_onehot, gl_dep, mean


_TOPK_F1, _F2_ONEHOT, _GL_DEP, _MEAN = _make_calls()


@functools.cache
def _sc_gather_sum_call():
  return functools.partial(
      pl.kernel,
      out_type=jax.ShapeDtypeStruct((SC_NODES, E), jnp.float32),
      mesh=plsc.VectorSubcoreMesh(core_axis_name="c", subcore_axis_name="s"),
      scratch_types=[
          pltpu.VMEM((GROUPS, G * K), jnp.int32),
          pltpu.VMEM((2, G * K, E), jnp.float32),
          pltpu.VMEM((2, G, E), jnp.float32),
          pltpu.SemaphoreType.DMA,
          pltpu.SemaphoreType.DMA,
      ],
  )(_sc_gather_sum_body)


@jax.jit
def kernel(loc, depot, W_init, b_init, W_ne, b_ne, W_dep, b_dep,
           W_t1, b_t1, W_t2, b_t2):
  x = loc
  xT = jnp.transpose(x, (0, 2, 1))
  nb, f1 = _TOPK_F1(x, xT, x[0], W_t1, W_init.T, b_init[None, :],
                    W_ne.T, b_ne[None, :], b_t1[None, :])
  gl, dep = _GL_DEP(f1[0], W_t2, b_t2[None, :], depot[:, 0, :], W_dep,
                    b_dep[None, :])
  nbf = nb.reshape(NODES, K)
  f2b = _sc_gather_sum_call()(nbf[NSPLIT:].reshape(NW, GROUPS, G * K), gl)
  f2a = _F2_ONEHOT(nbf, gl)
  f2 = jnp.concatenate([f2a, f2b], axis=0).reshape(B, N, E)
  mean = _MEAN(f2, dep)
  h = jnp.concatenate([dep[:, None, :], f2], axis=1)
  return (h, mean)


# speedup vs baseline: 13.0054x; 1.0630x over previous
import functools

import jax
import jax.numpy as jnp
from jax import lax
from jax.experimental import pallas as pl
from jax.experimental.pallas import tpu as pltpu
from jax.experimental.pallas import tpu_sc as plsc

B, N, D, E = 2, 2048, 2, 256
K = 10
R = 256
NC, NS = 2, 16
NW = NC * NS
NODES = B * N
NSPLIT = 3072
T_E = 256
SC_NODES = NODES - NSPLIT
NPW_SC = SC_NODES // NW
G = 8
GROUPS = NPW_SC // G


def _leaky(z):
  return jnp.where(z >= 0, z, 0.01 * z)


def _topk_f1_body(x_ref, xT_ref, x0_ref, wt1_ref, winitT_ref, binit_ref,
                  wneT_ref, bne_ref, bt1_ref, nb_ref, f1_ref):
  xi_x = x_ref[0, :, 0:1]
  xi_y = x_ref[0, :, 1:2]
  xj_x = xT_ref[0, 0:1, :]
  xj_y = xT_ref[0, 1:2, :]
  dx = xj_x - xi_x
  dy = xj_y - xi_y
  key = jnp.sqrt(dx * dx + dy * dy)
  fiota = lax.broadcasted_iota(jnp.int32, (R, N), 1).astype(jnp.float32)

  dn = (((1,), (1,)), ((), ()))
  wc0T = lax.dot_general(winitT_ref[...], wt1_ref[...], dn,
                         preferred_element_type=jnp.float32)
  bc0 = lax.dot_general(binit_ref[...], wt1_ref[...], dn,
                        preferred_element_type=jnp.float32) + bt1_ref[...]
  wcnT = lax.dot_general(wneT_ref[...], wt1_ref[...], dn,
                         preferred_element_type=jnp.float32)
  bcn = lax.dot_general(bne_ref[...], wt1_ref[...], dn,
                        preferred_element_type=jnp.float32) + bt1_ref[...]

  z0 = xi_x * wc0T[0:1, :] + xi_y * wc0T[1:2, :] + bc0
  acc = _leaky(z0)

  cols = []
  for _ in range(K):
    m = jnp.min(key, axis=1, keepdims=True)
    idx = jnp.min(jnp.where(key == m, fiota, float(N)), axis=1, keepdims=True)
    cols.append(idx)
    onehot_f = jnp.where(fiota == idx, 1.0, 0.0)
    key = jnp.where(fiota == idx, jnp.inf, key)
    nbxy = lax.dot_general(onehot_f, x0_ref[...],
                           (((1,), (0,)), ((), ())),
                           preferred_element_type=jnp.float32)
    dxn = nbxy[:, 0:1] - xi_x
    dyn = nbxy[:, 1:2] - xi_y
    acc = acc + _leaky(dxn * wcnT[0:1, :] + dyn * wcnT[1:2, :] + bcn)

  nb_ref[0] = jnp.concatenate(cols, axis=1).astype(jnp.int32)
  f1_ref[0] = acc


def _gl_dep_body(f1_ref, wt2_ref, bt2_ref, depot_ref, wdep_ref, bdep_ref,
                 gl_ref, dep_ref):
  dn = (((1,), (1,)), ((), ()))
  g = lax.dot_general(f1_ref[...], wt2_ref[...], dn,
                      preferred_element_type=jnp.float32) + bt2_ref[...]
  gl_ref[...] = _leaky(g)
  dep_ref[...] = lax.dot_general(depot_ref[...], wdep_ref[...], dn,
                                 preferred_element_type=jnp.float32) + bdep_ref[...]


def _f2_onehot_body(nbf_ref, gl_ref, out_ref):
  nbk = nbf_ref[...]
  iota = lax.broadcasted_iota(jnp.int32, (T_E, N), 1)
  mask = jnp.where(iota == nbk[:, 0:1], 1.0, 0.0)
  for k in range(1, K):
    mask = mask + jnp.where(iota == nbk[:, k:k + 1], 1.0, 0.0)
  out_ref[...] = lax.dot_general(mask, gl_ref[...], (((1,), (0,)), ((), ())),
                                 preferred_element_type=jnp.float32)


def _sc_gather_sum_body(idx_hbm, table_hbm, out_hbm, idx_v, rows_v, out_v,
                        gsem, osem):
  wid = lax.axis_index("s") * NC + lax.axis_index("c")
  base_node = wid * NPW_SC

  pltpu.sync_copy(idx_hbm.at[wid], idx_v)
  pltpu.async_copy(table_hbm.at[idx_v.at[0]], rows_v.at[0], gsem)

  def pair(i, carry):
    for b in range(2):
      g = 2 * i + b
      nxt = g + 1
      pltpu.make_async_copy(table_hbm.at[idx_v.at[g]], rows_v.at[b],
                            gsem).wait()

      @pl.when(nxt < GROUPS)
      def _():
        pltpu.async_copy(table_hbm.at[idx_v.at[nxt]], rows_v.at[1 - b], gsem)

      @pl.when(g >= 2)
      def _():
        pltpu.make_async_copy(out_v.at[b], out_hbm.at[pl.ds(base_node, G)],
                              osem).wait()

      for i2 in range(G):
        for c in range(E // 16):
          r = [rows_v[b, i2 * K + k, pl.ds(c * 16, 16)] for k in range(K)]
          s01, s23 = r[0] + r[1], r[2] + r[3]
          s45, s67 = r[4] + r[5], r[6] + r[7]
          s89 = r[8] + r[9]
          out_v[b, i2, pl.ds(c * 16, 16)] = ((s01 + s23) + (s45 + s67)) + s89
      pltpu.async_copy(out_v.at[b], out_hbm.at[pl.ds(base_node + g * G, G)],
                       osem)
    return carry

  lax.fori_loop(0, GROUPS // 2, pair, 0)
  for b in range(2):
    pltpu.make_async_copy(out_v.at[b], out_hbm.at[pl.ds(base_node, G)],
                          osem).wait()


def _mean_body(f2_ref, dep_ref, mean_ref):
  b = pl.program_id(0)
  s = jnp.sum(f2_ref[0], axis=0, keepdims=True) + dep_ref[pl.ds(b, 1), :]
  mean_ref[pl.ds(b, 1), :] = s / float(N + 1)


def _make_calls(interpret=False):
  topk_f1 = pl.pallas_call(
      _topk_f1_body,
      grid=(B, N // R),
      in_specs=[
          pl.BlockSpec((1, R, D), lambda b, t: (b, t, 0)),
          pl.BlockSpec((1, D, N), lambda b, t: (b, 0, 0)),
          pl.BlockSpec((N, D), lambda b, t: (0, 0)),
          pl.BlockSpec((E, E), lambda b, t: (0, 0)),
          pl.BlockSpec((D, E), lambda b, t: (0, 0)),
          pl.BlockSpec((1, E), lambda b, t: (0, 0)),
          pl.BlockSpec((D, E), lambda b, t: (0, 0)),
          pl.BlockSpec((1, E), lambda b, t: (0, 0)),
          pl.BlockSpec((1, E), lambda b, t: (0, 0)),
      ],
      out_specs=[
          pl.BlockSpec((1, R, K), lambda b, t: (b, t, 0)),
          pl.BlockSpec((1, R, E), lambda b, t: (b, t, 0)),
      ],
      out_shape=[
          jax.ShapeDtypeStruct((B, N, K), jnp.int32),
          jax.ShapeDtypeStruct((B, N, E), jnp.float32),
      ],
      interpret=interpret,
  )

  f2_onehot = pl.pallas_call(
      _f2_onehot_body,
      grid=(NSPLIT // T_E,),
      in_specs=[
          pl.BlockSpec((T_E, K), lambda t: (t, 0)),
          pl.BlockSpec((N, E), lambda t: (0, 0)),
      ],
      out_specs=pl.BlockSpec((T_E, E), lambda t: (t, 0)),
      out_shape=jax.ShapeDtypeStruct((NSPLIT, E), jnp.float32),
      interpret=interpret,
  )

  gl_dep = pl.pallas_call(
      _gl_dep_body,
      out_shape=[
          jax.ShapeDtypeStruct((N, E), jnp.float32),
          jax.ShapeDtypeStruct((B, E), jnp.float32),
      ],
      interpret=interpret,
  )

  mean = pl.pallas_call(
      _mean_body,
      grid=(B,),
      in_specs=[
          pl.BlockSpec((1, N, E), lambda b: (b, 0, 0)),
          pl.BlockSpec((B, E), lambda b: (0, 0)),
      ],
      out_specs=pl.BlockSpec((B, E), lambda b: (0, 0)),
      out_shape=jax.ShapeDtypeStruct((B, E), jnp.float32),
      interpret=interpret,
  )
  return topk_f1, f2_onehot, gl_dep, mean


_TOPK_F1, _F2_ONEHOT, _GL_DEP, _MEAN = _make_calls()


@functools.cache
def _sc_gather_sum_call():
  return functools.partial(
      pl.kernel,
      out_type=jax.ShapeDtypeStruct((SC_NODES, E), jnp.float32),
      mesh=plsc.VectorSubcoreMesh(core_axis_name="c", subcore_axis_name="s"),
      scratch_types=[
          pltpu.VMEM((GROUPS, G * K), jnp.int32),
          pltpu.VMEM((2, G * K, E), jnp.float32),
          pltpu.VMEM((2, G, E), jnp.float32),
          pltpu.SemaphoreType.DMA,
          pltpu.SemaphoreType.DMA,
      ],
  )(_sc_gather_sum_body)


@jax.jit
def kernel(loc, depot, W_init, b_init, W_ne, b_ne, W_dep, b_dep,
           W_t1, b_t1, W_t2, b_t2):
  x = loc
  xT = jnp.transpose(x, (0, 2, 1))
  nb, f1 = _TOPK_F1(x, xT, x[0], W_t1, W_init.T, b_init[None, :],
                    W_ne.T, b_ne[None, :], b_t1[None, :])
  gl, dep = _GL_DEP(f1[0], W_t2, b_t2[None, :], depot[:, 0, :], W_dep,
                    b_dep[None, :])
  nbf = nb.reshape(NODES, K)
  f2b = _sc_gather_sum_call()(nbf[NSPLIT:].reshape(NW, GROUPS, G * K), gl)
  f2a = _F2_ONEHOT(nbf, gl)
  f2 = jnp.concatenate([f2a, f2b], axis=0).reshape(B, N, E)
  mean = _MEAN(f2, dep)
  h = jnp.concatenate([dep[:, None, :], f2], axis=1)
  return (h, mean)
